# baseline (speedup 1.0000x reference)
import functools

import jax
import jax.numpy as jnp
from jax import lax
from jax.experimental import pallas as pl
from jax.experimental.pallas import tpu as pltpu
from jax.experimental.pallas import tpu_sc as plsc

N = 2048
D_IN = 1024
D_HID = 2048
D_OUT = 1024
E = 8
B = 128
NS = N + E * B
NB = NS // B
NB_PAD = 32
RING = 2
NW = 32
TPW = N // NW
CH = 128
NCH = N // CH
LCH = 1024

_f32 = jnp.float32
_i32 = jnp.int32


def _router_body(x_ref, wr_ref, br_ref, slot_ref, sc_ref, tos_ref, meta_ref,
                 o_scr, r_scr):
    x = x_ref[...]
    logits = jnp.dot(x, wr_ref[...], preferred_element_type=_f32) + br_ref[...]
    m = jnp.max(logits, axis=1, keepdims=True)
    ssum = jnp.sum(jnp.exp(logits - m), axis=1, keepdims=True)
    sig = 1.0 / ssum
    lane = lax.broadcasted_iota(_i32, (N, E), 1)
    eidx = jnp.min(jnp.where(logits >= m, lane, E), axis=1, keepdims=True)
    onehot = (lane == eidx).astype(_f32)
    o_scr[...] = onehot.reshape(NCH, CH, E)
    ltri = (lax.broadcasted_iota(_i32, (CH, CH), 0)
            >= lax.broadcasted_iota(_i32, (CH, CH), 1)).astype(_f32)

    def body(c, carry):
        ch = o_scr[c]
        cum = jnp.dot(ltri, ch, preferred_element_type=_f32)
        r_scr[c] = cum + carry - ch
        return carry + jnp.sum(ch, axis=0, keepdims=True)

    counts = lax.fori_loop(0, NCH, body, jnp.zeros((1, E), _f32))
    ranks = jnp.sum(r_scr[...].reshape(N, E) * onehot, axis=1, keepdims=True)
    pcounts = jnp.floor((counts + (B - 1)) * (1.0 / B)) * B
    strict = (lax.broadcasted_iota(_i32, (E, E), 0)
              < lax.broadcasted_iota(_i32, (E, E), 1)).astype(_f32)
    offs = jnp.dot(counts, strict, preferred_element_type=_f32,
                   precision=lax.Precision.HIGHEST)
    poffs = jnp.dot(pcounts, strict, preferred_element_type=_f32,
                    precision=lax.Precision.HIGHEST)
    slot = ranks + jnp.sum(onehot * poffs, axis=1, keepdims=True)
    pos = ranks + jnp.sum(onehot * offs, axis=1, keepdims=True)
    slot_i = slot.astype(_i32)
    pos_i = pos.astype(_i32)
    slot_ref[...] = slot_i
    g_parts = []
    for c in range(N // LCH):
        lane_t = lax.broadcasted_iota(_i32, (N, LCH), 1) + c * LCH
        pt = pos_i == lane_t
        g_parts.append(jnp.sum(jnp.where(pt, sig, 0.0), axis=0, keepdims=True))
    g_row = jnp.concatenate(g_parts, axis=1)
    tok_row = lax.broadcasted_iota(_i32, (1, N), 1).astype(_f32)
    sub8 = lax.broadcasted_iota(_i32, (8, N), 0)
    stacked = jnp.where(sub8 == 0, g_row, jnp.where(sub8 == 1, tok_row, 0.0))
    for c in range(NS // LCH):
        lane_s = lax.broadcasted_iota(_i32, (N, LCH), 1) + c * LCH
        qt = (slot_i == lane_s).astype(_f32)
        res = jnp.dot(stacked, qt, preferred_element_type=_f32,
                      precision=lax.Precision.HIGHEST)
        sc_ref[0:1, c * LCH:(c + 1) * LCH] = res[0:1]
        tos_ref[0:1, c * LCH:(c + 1) * LCH] = res[1:2].astype(_i32)
    hi = lax.Precision.HIGHEST
    ones_col = jnp.ones((E, 1), _f32)
    ones_row = jnp.ones((1, E), _f32)
    eye = (lax.broadcasted_iota(_i32, (E, E), 0)
           == lax.broadcasted_iota(_i32, (E, E), 1)).astype(_f32)

    def to_col(row):
        return jnp.sum(jnp.dot(ones_col, row, preferred_element_type=_f32,
                               precision=hi) * eye, axis=1, keepdims=True)

    pend_s = to_col(poffs + pcounts)
    bcol = (lax.broadcasted_iota(_i32, (E, NB_PAD), 1) * B).astype(_f32)
    indic = (bcol >= pend_s).astype(_f32)
    be = jnp.dot(ones_row, indic, preferred_element_type=_f32, precision=hi)
    be = jnp.minimum(be, float(E - 1))
    onehot_be = ((lax.broadcasted_iota(_i32, (E, NB_PAD), 0)).astype(_f32)
                 == jnp.dot(ones_col, be, preferred_element_type=_f32,
                            precision=hi)).astype(_f32)
    poffs_lk = jnp.dot(poffs, onehot_be, preferred_element_type=_f32,
                       precision=hi)
    pc_lk = jnp.dot(pcounts, onehot_be, preferred_element_type=_f32,
                    precision=hi)
    brow = (lax.broadcasted_iota(_i32, (1, NB_PAD), 1) * B).astype(_f32)
    isfirst = jnp.logical_and(brow == poffs_lk, pc_lk > 0.0)
    present = (pcounts > 0.0).astype(_f32)
    k_rank = jnp.dot(present, strict, preferred_element_type=_f32,
                     precision=hi)
    k_of_b = jnp.dot(k_rank, onehot_be, preferred_element_type=_f32,
                     precision=hi)

    def mod_ring(v):
        return v - RING * jnp.floor(v * (1.0 / RING))

    rk = jnp.logical_and(to_col(k_rank) == lax.broadcasted_iota(
        _i32, (E, E), 1).astype(_f32), to_col(present) > 0.0).astype(_f32)
    e_of_rank = jnp.dot((lax.broadcasted_iota(_i32, (1, E), 1)).astype(_f32),
                        rk, preferred_element_type=_f32, precision=hi)
    valid_rank = jnp.dot(ones_row, rk, preferred_element_type=_f32,
                         precision=hi)
    rb = ((lax.broadcasted_iota(_i32, (E, NB_PAD), 0)).astype(_f32)
          == jnp.dot(ones_col, k_of_b + (RING - 1.0),
                     preferred_element_type=_f32,
                     precision=hi)).astype(_f32)
    fetch_e = jnp.dot(e_of_rank, rb, preferred_element_type=_f32, precision=hi)
    fetch_v = jnp.dot(valid_rank, rb, preferred_element_type=_f32,
                      precision=hi)
    fetch_eid = jnp.where(fetch_v > 0.0, fetch_e, -1.0)
    fetch_slot = mod_ring(k_of_b + (RING - 1.0))
    lane_r = lax.broadcasted_iota(_i32, (1, NB_PAD), 1).astype(_f32)
    row5 = jnp.full((1, NB_PAD), -1.0, _f32)
    for r in range(RING - 1):
        er = jnp.sum(e_of_rank * (lax.broadcasted_iota(_i32, (1, E), 1) == r),
                     axis=1, keepdims=True)
        vr = jnp.sum(valid_rank * (lax.broadcasted_iota(_i32, (1, E), 1) == r),
                     axis=1, keepdims=True)
        row5 = jnp.where(lane_r == r, jnp.where(vr > 0.0, er, -1.0), row5)
    meta_ref[0:1, :] = be.astype(_i32)
    meta_ref[1:2, :] = isfirst.astype(_i32)
    meta_ref[2:3, :] = mod_ring(k_of_b).astype(_i32)
    meta_ref[3:4, :] = fetch_eid.astype(_i32)
    meta_ref[4:5, :] = fetch_slot.astype(_i32)
    meta_ref[5:6, :] = row5.astype(_i32)
    meta_ref[6:8, :] = jnp.zeros((2, NB_PAD), _i32)


def _router_call(x, wr, br2):
    return pl.pallas_call(
        _router_body,
        out_shape=[
            jax.ShapeDtypeStruct((N, 1), _i32),
            jax.ShapeDtypeStruct((1, NS), _f32),
            jax.ShapeDtypeStruct((1, NS), _i32),
            jax.ShapeDtypeStruct((8, NB_PAD), _i32),
        ],
        scratch_shapes=[
            pltpu.VMEM((NCH, CH, E), _f32),
            pltpu.VMEM((NCH, CH, E), _f32),
        ],
    )(x, wr, br2)


def _ffn_body(meta_ref, x_ref, tos_ref, w1_hbm, b1_ref, w2_hbm, b2_ref,
              sc_ref, out_ref, w1buf, w2buf, sems):
    b = pl.program_id(0)
    kslot = meta_ref[2, b]

    def issue(e, s):
        pltpu.make_async_copy(w1_hbm.at[e], w1buf.at[s], sems.at[s, 0]).start()
        pltpu.make_async_copy(w2_hbm.at[e], w2buf.at[s], sems.at[s, 1]).start()

    @pl.when(b == 0)
    def _():
        issue(meta_ref[5, 0], 0)
        for r in range(1, RING - 1):
            er = meta_ref[5, r]

            @pl.when(er >= 0)
            def _():
                issue(er, r)

    @pl.when(meta_ref[1, b] == 1)
    def _():
        fetch_eid = meta_ref[3, b]

        @pl.when(fetch_eid >= 0)
        def _():
            issue(fetch_eid, meta_ref[4, b])

        eb = meta_ref[0, b]
        pltpu.make_async_copy(w1_hbm.at[eb], w1buf.at[kslot],
                              sems.at[kslot, 0]).wait()
        pltpu.make_async_copy(w2_hbm.at[eb], w2buf.at[kslot],
                              sems.at[kslot, 1]).wait()

    perm = (lax.broadcasted_iota(_i32, (B, N), 1) == tos_ref[...]).astype(_f32)
    xb = jnp.dot(perm, x_ref[...], preferred_element_type=_f32)
    h = jnp.dot(xb, w1buf[kslot], preferred_element_type=_f32) + b1_ref[0]
    h = jnp.maximum(h, 0.0)
    y = jnp.dot(h, w2buf[kslot], preferred_element_type=_f32) + b2_ref[0]
    out_ref[...] = y * sc_ref[...]


def _ffn_call(meta, x, tos2, w1, b1, w2, b2, sc2):
    grid_spec = pltpu.PrefetchScalarGridSpec(
        num_scalar_prefetch=1,
        grid=(NB,),
        in_specs=[
            pl.BlockSpec((N, D_IN), lambda b, m: (0, 0)),
            pl.BlockSpec((B, 1), lambda b, m: (b, 0)),
            pl.BlockSpec(memory_space=pl.ANY),
            pl.BlockSpec((1, 1, D_HID), lambda b, m: (m[0, b], 0, 0)),
            pl.BlockSpec(memory_space=pl.ANY),
            pl.BlockSpec((1, 1, D_OUT), lambda b, m: (m[0, b], 0, 0)),
            pl.BlockSpec((B, 1), lambda b, m: (b, 0)),
        ],
        out_specs=pl.BlockSpec((B, D_OUT), lambda b, m: (b, 0)),
        scratch_shapes=[
            pltpu.VMEM((RING, D_IN, D_HID), _f32),
            pltpu.VMEM((RING, D_HID, D_OUT), _f32),
            pltpu.SemaphoreType.DMA((RING, 2)),
        ],
    )
    return pl.pallas_call(
        _ffn_body,
        grid_spec=grid_spec,
        out_shape=jax.ShapeDtypeStruct((NS, D_OUT), _f32),
    )(meta, x, tos2, w1, b1.reshape(E, 1, D_HID), w2,
      b2.reshape(E, 1, D_OUT), sc2)


def _combine_call(ys, slot):
    mesh = plsc.VectorSubcoreMesh(core_axis_name="c", subcore_axis_name="s")

    @functools.partial(
        pl.kernel,
        mesh=mesh,
        out_type=jax.ShapeDtypeStruct((N, D_OUT), _f32),
        scratch_types=[
            pltpu.VMEM((TPW,), _i32),
            pltpu.VMEM((TPW, D_OUT), _f32),
            pltpu.SemaphoreType.DMA,
        ],
    )
    def k4(ys_hbm, slot_hbm, out_hbm, idx_v, rows_v, sem):
        wid = lax.axis_index("s") * 2 + lax.axis_index("c")
        base = wid * TPW
        pltpu.sync_copy(slot_hbm.at[pl.ds(base, TPW)], idx_v)
        pltpu.async_copy(ys_hbm.at[idx_v], rows_v, sem).wait()
        pltpu.sync_copy(rows_v, out_hbm.at[pl.ds(base, TPW)])

    return k4(ys, slot)


def kernel(x, Wr, br, W1, b1, W2, b2):
    slot2, sc_row, tos_row, meta = _router_call(x, Wr, br.reshape(1, E))
    ys = _ffn_call(meta, x, tos_row.reshape(NS, 1), W1, b1, W2, b2,
                   sc_row.reshape(NS, 1))
    return _combine_call(ys, slot2.reshape(N))


# speedup vs baseline: 1.0322x; 1.0322x over previous
import functools

import jax
import jax.numpy as jnp
from jax import lax
from jax.experimental import pallas as pl
from jax.experimental.pallas import tpu as pltpu
from jax.experimental.pallas import tpu_sc as plsc

N = 2048
D_IN = 1024
D_HID = 2048
D_OUT = 1024
E = 8
B = 128
NS = N + E * B
NB = NS // B
NB_PAD = 32
RING = 2
NW = 32
TPW = N // NW
CH = 128
NCH = N // CH
LCH = 1024

_f32 = jnp.float32
_i32 = jnp.int32


def _router_body(x_ref, wr_ref, br_ref, slot_ref, sc_ref, tos_ref, meta_ref,
                 o_scr, r_scr):
    x = x_ref[...]
    logits = jnp.dot(x, wr_ref[...], preferred_element_type=_f32) + br_ref[...]
    m = jnp.max(logits, axis=1, keepdims=True)
    ssum = jnp.sum(jnp.exp(logits - m), axis=1, keepdims=True)
    sig = 1.0 / ssum
    lane = lax.broadcasted_iota(_i32, (N, E), 1)
    eidx = jnp.min(jnp.where(logits >= m, lane, E), axis=1, keepdims=True)
    onehot = (lane == eidx).astype(_f32)
    o_scr[...] = onehot.reshape(NCH, CH, E)
    ltri = (lax.broadcasted_iota(_i32, (CH, CH), 0)
            >= lax.broadcasted_iota(_i32, (CH, CH), 1)).astype(_f32)

    def body(c, carry):
        ch = o_scr[c]
        cum = jnp.dot(ltri, ch, preferred_element_type=_f32)
        r_scr[c] = cum + carry - ch
        return carry + jnp.sum(ch, axis=0, keepdims=True)

    counts = lax.fori_loop(0, NCH, body, jnp.zeros((1, E), _f32))
    ranks = jnp.sum(r_scr[...].reshape(N, E) * onehot, axis=1, keepdims=True)
    pcounts = jnp.floor((counts + (B - 1)) * (1.0 / B)) * B
    strict = (lax.broadcasted_iota(_i32, (E, E), 0)
              < lax.broadcasted_iota(_i32, (E, E), 1)).astype(_f32)
    offs = jnp.dot(counts, strict, preferred_element_type=_f32,
                   precision=lax.Precision.HIGHEST)
    poffs = jnp.dot(pcounts, strict, preferred_element_type=_f32,
                    precision=lax.Precision.HIGHEST)
    slot = ranks + jnp.sum(onehot * poffs, axis=1, keepdims=True)
    pos = ranks + jnp.sum(onehot * offs, axis=1, keepdims=True)
    slot_i = slot.astype(_i32)
    pos_i = pos.astype(_i32)
    slot_ref[...] = slot_i
    g_parts = []
    for c in range(N // LCH):
        lane_t = lax.broadcasted_iota(_i32, (N, LCH), 1) + c * LCH
        pt = pos_i == lane_t
        g_parts.append(jnp.sum(jnp.where(pt, sig, 0.0), axis=0, keepdims=True))
    g_row = jnp.concatenate(g_parts, axis=1)
    tok_row = lax.broadcasted_iota(_i32, (1, N), 1).astype(_f32)
    sub8 = lax.broadcasted_iota(_i32, (8, N), 0)
    stacked = jnp.where(sub8 == 0, g_row, jnp.where(sub8 == 1, tok_row, 0.0))
    for c in range(NS // LCH):
        lane_s = lax.broadcasted_iota(_i32, (N, LCH), 1) + c * LCH
        qt = (slot_i == lane_s).astype(_f32)
        res = jnp.dot(stacked, qt, preferred_element_type=_f32,
                      precision=lax.Precision.HIGHEST)
        sc_ref[0:1, c * LCH:(c + 1) * LCH] = res[0:1]
        tos_ref[0:1, c * LCH:(c + 1) * LCH] = res[1:2].astype(_i32)
    hi = lax.Precision.HIGHEST
    ones_col = jnp.ones((E, 1), _f32)
    ones_row = jnp.ones((1, E), _f32)
    eye = (lax.broadcasted_iota(_i32, (E, E), 0)
           == lax.broadcasted_iota(_i32, (E, E), 1)).astype(_f32)

    def to_col(row):
        return jnp.sum(jnp.dot(ones_col, row, preferred_element_type=_f32,
                               precision=hi) * eye, axis=1, keepdims=True)

    pend_s = to_col(poffs + pcounts)
    bcol = (lax.broadcasted_iota(_i32, (E, NB_PAD), 1) * B).astype(_f32)
    indic = (bcol >= pend_s).astype(_f32)
    be = jnp.dot(ones_row, indic, preferred_element_type=_f32, precision=hi)
    be = jnp.minimum(be, float(E - 1))
    onehot_be = ((lax.broadcasted_iota(_i32, (E, NB_PAD), 0)).astype(_f32)
                 == jnp.dot(ones_col, be, preferred_element_type=_f32,
                            precision=hi)).astype(_f32)
    poffs_lk = jnp.dot(poffs, onehot_be, preferred_element_type=_f32,
                       precision=hi)
    pc_lk = jnp.dot(pcounts, onehot_be, preferred_element_type=_f32,
                    precision=hi)
    brow = (lax.broadcasted_iota(_i32, (1, NB_PAD), 1) * B).astype(_f32)
    isfirst = jnp.logical_and(brow == poffs_lk, pc_lk > 0.0)
    present = (pcounts > 0.0).astype(_f32)
    k_rank = jnp.dot(present, strict, preferred_element_type=_f32,
                     precision=hi)
    k_of_b = jnp.dot(k_rank, onehot_be, preferred_element_type=_f32,
                     precision=hi)

    def mod_ring(v):
        return v - RING * jnp.floor(v * (1.0 / RING))

    rk = jnp.logical_and(to_col(k_rank) == lax.broadcasted_iota(
        _i32, (E, E), 1).astype(_f32), to_col(present) > 0.0).astype(_f32)
    e_of_rank = jnp.dot((lax.broadcasted_iota(_i32, (1, E), 1)).astype(_f32),
                        rk, preferred_element_type=_f32, precision=hi)
    valid_rank = jnp.dot(ones_row, rk, preferred_element_type=_f32,
                         precision=hi)
    rb = ((lax.broadcasted_iota(_i32, (E, NB_PAD), 0)).astype(_f32)
          == jnp.dot(ones_col, k_of_b + (RING - 1.0),
                     preferred_element_type=_f32,
                     precision=hi)).astype(_f32)
    fetch_e = jnp.dot(e_of_rank, rb, preferred_element_type=_f32, precision=hi)
    fetch_v = jnp.dot(valid_rank, rb, preferred_element_type=_f32,
                      precision=hi)
    fetch_eid = jnp.where(fetch_v > 0.0, fetch_e, -1.0)
    fetch_slot = mod_ring(k_of_b + (RING - 1.0))
    lane_r = lax.broadcasted_iota(_i32, (1, NB_PAD), 1).astype(_f32)
    row5 = jnp.full((1, NB_PAD), -1.0, _f32)
    for r in range(RING - 1):
        er = jnp.sum(e_of_rank * (lax.broadcasted_iota(_i32, (1, E), 1) == r),
                     axis=1, keepdims=True)
        vr = jnp.sum(valid_rank * (lax.broadcasted_iota(_i32, (1, E), 1) == r),
                     axis=1, keepdims=True)
        row5 = jnp.where(lane_r == r, jnp.where(vr > 0.0, er, -1.0), row5)
    total_used = jnp.max(poffs + pcounts, axis=1, keepdims=True)
    active = (brow < total_used)
    meta_ref[0:1, :] = be.astype(_i32)
    meta_ref[1:2, :] = isfirst.astype(_i32)
    meta_ref[2:3, :] = mod_ring(k_of_b).astype(_i32)
    meta_ref[3:4, :] = fetch_eid.astype(_i32)
    meta_ref[4:5, :] = fetch_slot.astype(_i32)
    meta_ref[5:6, :] = row5.astype(_i32)
    meta_ref[6:7, :] = active.astype(_i32)
    meta_ref[7:8, :] = jnp.zeros((1, NB_PAD), _i32)


def _router_call(x, wr, br2):
    return pl.pallas_call(
        _router_body,
        out_shape=[
            jax.ShapeDtypeStruct((N, 1), _i32),
            jax.ShapeDtypeStruct((1, NS), _f32),
            jax.ShapeDtypeStruct((1, NS), _i32),
            jax.ShapeDtypeStruct((8, NB_PAD), _i32),
        ],
        scratch_shapes=[
            pltpu.VMEM((NCH, CH, E), _f32),
            pltpu.VMEM((NCH, CH, E), _f32),
        ],
    )(x, wr, br2)


def _ffn_body(meta_ref, x_ref, tos_ref, w1_hbm, b1_ref, w2_hbm, b2_ref,
              sc_ref, out_ref, w1buf, w2buf, sems):
    b = pl.program_id(0)
    kslot = meta_ref[2, b]

    def issue(e, s):
        pltpu.make_async_copy(w1_hbm.at[e], w1buf.at[s], sems.at[s, 0]).start()
        pltpu.make_async_copy(w2_hbm.at[e], w2buf.at[s], sems.at[s, 1]).start()

    @pl.when(b == 0)
    def _():
        issue(meta_ref[5, 0], 0)
        for r in range(1, RING - 1):
            er = meta_ref[5, r]

            @pl.when(er >= 0)
            def _():
                issue(er, r)

    @pl.when(meta_ref[1, b] == 1)
    def _():
        fetch_eid = meta_ref[3, b]

        @pl.when(fetch_eid >= 0)
        def _():
            issue(fetch_eid, meta_ref[4, b])

        eb = meta_ref[0, b]
        pltpu.make_async_copy(w1_hbm.at[eb], w1buf.at[kslot],
                              sems.at[kslot, 0]).wait()
        pltpu.make_async_copy(w2_hbm.at[eb], w2buf.at[kslot],
                              sems.at[kslot, 1]).wait()

    @pl.when(meta_ref[6, b] == 1)
    def _():
        perm = (lax.broadcasted_iota(_i32, (B, N), 1)
                == tos_ref[...]).astype(_f32)
        xb = jnp.dot(perm, x_ref[...], preferred_element_type=_f32)
        h = jnp.dot(xb, w1buf[kslot], preferred_element_type=_f32) + b1_ref[0]
        h = jnp.maximum(h, 0.0)
        y = jnp.dot(h, w2buf[kslot], preferred_element_type=_f32) + b2_ref[0]
        out_ref[...] = y * sc_ref[...]


def _ffn_call(meta, x, tos2, w1, b1, w2, b2, sc2):
    grid_spec = pltpu.PrefetchScalarGridSpec(
        num_scalar_prefetch=1,
        grid=(NB,),
        in_specs=[
            pl.BlockSpec((N, D_IN), lambda b, m: (0, 0)),
            pl.BlockSpec((B, 1), lambda b, m: (b, 0)),
            pl.BlockSpec(memory_space=pl.ANY),
            pl.BlockSpec((1, 1, D_HID), lambda b, m: (m[0, b], 0, 0)),
            pl.BlockSpec(memory_space=pl.ANY),
            pl.BlockSpec((1, 1, D_OUT), lambda b, m: (m[0, b], 0, 0)),
            pl.BlockSpec((B, 1), lambda b, m: (b, 0)),
        ],
        out_specs=pl.BlockSpec((B, D_OUT), lambda b, m: (b, 0)),
        scratch_shapes=[
            pltpu.VMEM((RING, D_IN, D_HID), _f32),
            pltpu.VMEM((RING, D_HID, D_OUT), _f32),
            pltpu.SemaphoreType.DMA((RING, 2)),
        ],
    )
    return pl.pallas_call(
        _ffn_body,
        grid_spec=grid_spec,
        out_shape=jax.ShapeDtypeStruct((NS, D_OUT), _f32),
    )(meta, x, tos2, w1, b1.reshape(E, 1, D_HID), w2,
      b2.reshape(E, 1, D_OUT), sc2)


def _combine_call(ys, slot):
    mesh = plsc.VectorSubcoreMesh(core_axis_name="c", subcore_axis_name="s")

    @functools.partial(
        pl.kernel,
        mesh=mesh,
        out_type=jax.ShapeDtypeStruct((N, D_OUT), _f32),
        scratch_types=[
            pltpu.VMEM((TPW,), _i32),
            pltpu.VMEM((TPW, D_OUT), _f32),
            pltpu.SemaphoreType.DMA,
        ],
    )
    def k4(ys_hbm, slot_hbm, out_hbm, idx_v, rows_v, sem):
        wid = lax.axis_index("s") * 2 + lax.axis_index("c")
        base = wid * TPW
        pltpu.sync_copy(slot_hbm.at[pl.ds(base, TPW)], idx_v)
        pltpu.async_copy(ys_hbm.at[idx_v], rows_v, sem).wait()
        pltpu.sync_copy(rows_v, out_hbm.at[pl.ds(base, TPW)])

    return k4(ys, slot)


def kernel(x, Wr, br, W1, b1, W2, b2):
    slot2, sc_row, tos_row, meta = _router_call(x, Wr, br.reshape(1, E))
    ys = _ffn_call(meta, x, tos_row.reshape(NS, 1), W1, b1, W2, b2,
                   sc_row.reshape(NS, 1))
    return _combine_call(ys, slot2.reshape(N))


# speedup vs baseline: 1.1240x; 1.0889x over previous
import functools

import jax
import jax.numpy as jnp
from jax import lax
from jax.experimental import pallas as pl
from jax.experimental.pallas import tpu as pltpu
from jax.experimental.pallas import tpu_sc as plsc

N = 2048
D_IN = 1024
D_HID = 2048
D_OUT = 1024
E = 8
B = 128
NS = N + E * B
NB = NS // B
NB_PAD = 32
RING = 2
NW = 32
TPW = N // NW
CH = 128
NCH = N // CH
LCH = 1024

_f32 = jnp.float32
_i32 = jnp.int32


def _router_body(x_ref, wr_ref, br_ref, slot_ref, sc_ref, tos_ref, meta_ref,
                 o_scr, r_scr):
    x = x_ref[...]
    logits = jnp.dot(x, wr_ref[...], preferred_element_type=_f32) + br_ref[...]
    m = jnp.max(logits, axis=1, keepdims=True)
    ssum = jnp.sum(jnp.exp(logits - m), axis=1, keepdims=True)
    sig = 1.0 / ssum
    lane = lax.broadcasted_iota(_i32, (N, E), 1)
    eidx = jnp.min(jnp.where(logits >= m, lane, E), axis=1, keepdims=True)
    onehot = (lane == eidx).astype(_f32)
    o_scr[...] = onehot.reshape(NCH, CH, E)
    ltri = (lax.broadcasted_iota(_i32, (CH, CH), 0)
            >= lax.broadcasted_iota(_i32, (CH, CH), 1)).astype(_f32)

    def body(c, carry):
        ch = o_scr[c]
        cum = jnp.dot(ltri, ch, preferred_element_type=_f32)
        r_scr[c] = cum + carry - ch
        return carry + jnp.sum(ch, axis=0, keepdims=True)

    counts = lax.fori_loop(0, NCH, body, jnp.zeros((1, E), _f32))
    ranks = jnp.sum(r_scr[...].reshape(N, E) * onehot, axis=1, keepdims=True)
    pcounts = jnp.floor((counts + (B - 1)) * (1.0 / B)) * B
    strict = (lax.broadcasted_iota(_i32, (E, E), 0)
              < lax.broadcasted_iota(_i32, (E, E), 1)).astype(_f32)
    offs = jnp.dot(counts, strict, preferred_element_type=_f32,
                   precision=lax.Precision.HIGHEST)
    poffs = jnp.dot(pcounts, strict, preferred_element_type=_f32,
                    precision=lax.Precision.HIGHEST)
    slot = ranks + jnp.sum(onehot * poffs, axis=1, keepdims=True)
    pos = ranks + jnp.sum(onehot * offs, axis=1, keepdims=True)
    slot_i = slot.astype(_i32)
    pos_i = pos.astype(_i32)
    slot_ref[...] = slot_i
    g_parts = []
    for c in range(N // LCH):
        lane_t = lax.broadcasted_iota(_i32, (N, LCH), 1) + c * LCH
        pt = pos_i == lane_t
        g_parts.append(jnp.sum(jnp.where(pt, sig, 0.0), axis=0, keepdims=True))
    g_row = jnp.concatenate(g_parts, axis=1)
    tok_row = lax.broadcasted_iota(_i32, (1, N), 1).astype(_f32)
    hi_row = jnp.floor(tok_row * (1.0 / 64.0))
    lo_row = tok_row - 64.0 * hi_row
    sub8 = lax.broadcasted_iota(_i32, (8, N), 0)
    stacked = jnp.where(sub8 == 0, g_row,
                        jnp.where(sub8 == 1, hi_row,
                                  jnp.where(sub8 == 2, lo_row, 0.0)))
    for c in range(NS // LCH):
        lane_s = lax.broadcasted_iota(_i32, (N, LCH), 1) + c * LCH
        qt = (slot_i == lane_s).astype(_f32)
        res = jnp.dot(stacked, qt, preferred_element_type=_f32)
        sc_ref[0:1, c * LCH:(c + 1) * LCH] = res[0:1]
        tos_ref[0:1, c * LCH:(c + 1) * LCH] = (
            64.0 * res[1:2] + res[2:3]).astype(_i32)
    hi = lax.Precision.HIGHEST
    ones_col = jnp.ones((E, 1), _f32)
    ones_row = jnp.ones((1, E), _f32)
    eye = (lax.broadcasted_iota(_i32, (E, E), 0)
           == lax.broadcasted_iota(_i32, (E, E), 1)).astype(_f32)

    def to_col(row):
        return jnp.sum(jnp.dot(ones_col, row, preferred_element_type=_f32,
                               precision=hi) * eye, axis=1, keepdims=True)

    pend_s = to_col(poffs + pcounts)
    bcol = (lax.broadcasted_iota(_i32, (E, NB_PAD), 1) * B).astype(_f32)
    indic = (bcol >= pend_s).astype(_f32)
    be = jnp.dot(ones_row, indic, preferred_element_type=_f32, precision=hi)
    be = jnp.minimum(be, float(E - 1))
    onehot_be = ((lax.broadcasted_iota(_i32, (E, NB_PAD), 0)).astype(_f32)
                 == jnp.dot(ones_col, be, preferred_element_type=_f32,
                            precision=hi)).astype(_f32)
    poffs_lk = jnp.dot(poffs, onehot_be, preferred_element_type=_f32,
                       precision=hi)
    pc_lk = jnp.dot(pcounts, onehot_be, preferred_element_type=_f32,
                    precision=hi)
    brow = (lax.broadcasted_iota(_i32, (1, NB_PAD), 1) * B).astype(_f32)
    isfirst = jnp.logical_and(brow == poffs_lk, pc_lk > 0.0)
    present = (pcounts > 0.0).astype(_f32)
    k_rank = jnp.dot(present, strict, preferred_element_type=_f32,
                     precision=hi)
    k_of_b = jnp.dot(k_rank, onehot_be, preferred_element_type=_f32,
                     precision=hi)

    def mod_ring(v):
        return v - RING * jnp.floor(v * (1.0 / RING))

    rk = jnp.logical_and(to_col(k_rank) == lax.broadcasted_iota(
        _i32, (E, E), 1).astype(_f32), to_col(present) > 0.0).astype(_f32)
    e_of_rank = jnp.dot((lax.broadcasted_iota(_i32, (1, E), 1)).astype(_f32),
                        rk, preferred_element_type=_f32, precision=hi)
    valid_rank = jnp.dot(ones_row, rk, preferred_element_type=_f32,
                         precision=hi)
    rb = ((lax.broadcasted_iota(_i32, (E, NB_PAD), 0)).astype(_f32)
          == jnp.dot(ones_col, k_of_b + (RING - 1.0),
                     preferred_element_type=_f32,
                     precision=hi)).astype(_f32)
    fetch_e = jnp.dot(e_of_rank, rb, preferred_element_type=_f32, precision=hi)
    fetch_v = jnp.dot(valid_rank, rb, preferred_element_type=_f32,
                      precision=hi)
    fetch_eid = jnp.where(fetch_v > 0.0, fetch_e, -1.0)
    fetch_slot = mod_ring(k_of_b + (RING - 1.0))
    lane_r = lax.broadcasted_iota(_i32, (1, NB_PAD), 1).astype(_f32)
    row5 = jnp.full((1, NB_PAD), -1.0, _f32)
    for r in range(RING - 1):
        er = jnp.sum(e_of_rank * (lax.broadcasted_iota(_i32, (1, E), 1) == r),
                     axis=1, keepdims=True)
        vr = jnp.sum(valid_rank * (lax.broadcasted_iota(_i32, (1, E), 1) == r),
                     axis=1, keepdims=True)
        row5 = jnp.where(lane_r == r, jnp.where(vr > 0.0, er, -1.0), row5)
    total_used = jnp.max(poffs + pcounts, axis=1, keepdims=True)
    active = (brow < total_used)
    meta_ref[0:1, :] = be.astype(_i32)
    meta_ref[1:2, :] = isfirst.astype(_i32)
    meta_ref[2:3, :] = mod_ring(k_of_b).astype(_i32)
    meta_ref[3:4, :] = fetch_eid.astype(_i32)
    meta_ref[4:5, :] = fetch_slot.astype(_i32)
    meta_ref[5:6, :] = row5.astype(_i32)
    meta_ref[6:7, :] = active.astype(_i32)
    meta_ref[7:8, :] = jnp.zeros((1, NB_PAD), _i32)


def _router_call(x, wr, br2):
    return pl.pallas_call(
        _router_body,
        out_shape=[
            jax.ShapeDtypeStruct((N, 1), _i32),
            jax.ShapeDtypeStruct((1, NS), _f32),
            jax.ShapeDtypeStruct((1, NS), _i32),
            jax.ShapeDtypeStruct((8, NB_PAD), _i32),
        ],
        scratch_shapes=[
            pltpu.VMEM((NCH, CH, E), _f32),
            pltpu.VMEM((NCH, CH, E), _f32),
        ],
    )(x, wr, br2)


def _ffn_body(meta_ref, x_ref, tos_ref, w1_hbm, b1_ref, w2_hbm, b2_ref,
              sc_ref, out_ref, w1buf, w2buf, sems):
    b = pl.program_id(0)
    kslot = meta_ref[2, b]

    def issue(e, s):
        pltpu.make_async_copy(w1_hbm.at[e], w1buf.at[s], sems.at[s, 0]).start()
        pltpu.make_async_copy(w2_hbm.at[e], w2buf.at[s], sems.at[s, 1]).start()

    @pl.when(b == 0)
    def _():
        issue(meta_ref[5, 0], 0)
        for r in range(1, RING - 1):
            er = meta_ref[5, r]

            @pl.when(er >= 0)
            def _():
                issue(er, r)

    @pl.when(meta_ref[1, b] == 1)
    def _():
        fetch_eid = meta_ref[3, b]

        @pl.when(fetch_eid >= 0)
        def _():
            issue(fetch_eid, meta_ref[4, b])

        eb = meta_ref[0, b]
        pltpu.make_async_copy(w1_hbm.at[eb], w1buf.at[kslot],
                              sems.at[kslot, 0]).wait()
        pltpu.make_async_copy(w2_hbm.at[eb], w2buf.at[kslot],
                              sems.at[kslot, 1]).wait()

    @pl.when(meta_ref[6, b] == 1)
    def _():
        perm = (lax.broadcasted_iota(_i32, (B, N), 1)
                == tos_ref[...]).astype(_f32)
        xb = jnp.dot(perm, x_ref[...], preferred_element_type=_f32)
        h = jnp.dot(xb, w1buf[kslot], preferred_element_type=_f32) + b1_ref[0]
        h = jnp.maximum(h, 0.0)
        y = jnp.dot(h, w2buf[kslot], preferred_element_type=_f32) + b2_ref[0]
        out_ref[...] = y * sc_ref[...]


def _ffn_call(meta, x, tos2, w1, b1, w2, b2, sc2):
    grid_spec = pltpu.PrefetchScalarGridSpec(
        num_scalar_prefetch=1,
        grid=(NB,),
        in_specs=[
            pl.BlockSpec((N, D_IN), lambda b, m: (0, 0)),
            pl.BlockSpec((B, 1), lambda b, m: (b, 0)),
            pl.BlockSpec(memory_space=pl.ANY),
            pl.BlockSpec((1, 1, D_HID), lambda b, m: (m[0, b], 0, 0)),
            pl.BlockSpec(memory_space=pl.ANY),
            pl.BlockSpec((1, 1, D_OUT), lambda b, m: (m[0, b], 0, 0)),
            pl.BlockSpec((B, 1), lambda b, m: (b, 0)),
        ],
        out_specs=pl.BlockSpec((B, D_OUT), lambda b, m: (b, 0)),
        scratch_shapes=[
            pltpu.VMEM((RING, D_IN, D_HID), _f32),
            pltpu.VMEM((RING, D_HID, D_OUT), _f32),
            pltpu.SemaphoreType.DMA((RING, 2)),
        ],
    )
    return pl.pallas_call(
        _ffn_body,
        grid_spec=grid_spec,
        out_shape=jax.ShapeDtypeStruct((NS, D_OUT), _f32),
    )(meta, x, tos2, w1, b1.reshape(E, 1, D_HID), w2,
      b2.reshape(E, 1, D_OUT), sc2)


def _combine_call(ys, slot):
    mesh = plsc.VectorSubcoreMesh(core_axis_name="c", subcore_axis_name="s")

    @functools.partial(
        pl.kernel,
        mesh=mesh,
        out_type=jax.ShapeDtypeStruct((N, D_OUT), _f32),
        scratch_types=[
            pltpu.VMEM((TPW,), _i32),
            pltpu.VMEM((TPW, D_OUT), _f32),
            pltpu.SemaphoreType.DMA,
        ],
    )
    def k4(ys_hbm, slot_hbm, out_hbm, idx_v, rows_v, sem):
        wid = lax.axis_index("s") * 2 + lax.axis_index("c")
        base = wid * TPW
        pltpu.sync_copy(slot_hbm.at[pl.ds(base, TPW)], idx_v)
        pltpu.async_copy(ys_hbm.at[idx_v], rows_v, sem).wait()
        pltpu.sync_copy(rows_v, out_hbm.at[pl.ds(base, TPW)])

    return k4(ys, slot)


def kernel(x, Wr, br, W1, b1, W2, b2):
    slot2, sc_row, tos_row, meta = _router_call(x, Wr, br.reshape(1, E))
    ys = _ffn_call(meta, x, tos_row.reshape(NS, 1), W1, b1, W2, b2,
                   sc_row.reshape(NS, 1))
    return _combine_call(ys, slot2.reshape(N))
